# baseline (speedup 1.0000x reference)
import functools
import math

import jax
import jax.numpy as jnp
from jax import lax
from jax.experimental import pallas as pl
from jax.experimental.pallas import tpu as pltpu
from jax.experimental.pallas import tpu_sc as plsc

_F = 16
_H = 512
_W = 512
_N = _H * _W
_GRID_N = (32, 64, 128, 256, 512)
_NUM_LOOKUPS = 9

_NW = 32
_CHUNK = 128
_QPR = _W // _CHUNK
_NCHUNKS = _N // _CHUNK
_PER_W = _NCHUNKS // _NW
_ROWS_W = _H // _NW
_SUPR = 2
_SUP = _SUPR * _QPR


def _idx_body(hc_ref, wc_ref, out_ref):
    hc = hc_ref[...]
    wc = wc_ref[...]
    n0 = _GRID_N[0]
    ih = (hc * float(n0 - 1)).astype(jnp.int32)
    iw = (wc * float(n0 - 1)).astype(jnp.int32)
    out_ref[0] = ih * n0 + iw
    off = n0 * n0
    k = 1
    for i in range(1, 5):
        n = _GRID_N[i]
        m = math.pi * (2.0 ** i)
        xh = hc * m
        xw = wc * m
        sh = jnp.abs(jnp.sin(xh))
        sw = jnp.abs(jnp.sin(xw))
        out_ref[k] = (off + (sh * float(n - 1)).astype(jnp.int32) * n
                      + (sw * float(n - 1)).astype(jnp.int32))
        k += 1
        ch = jnp.abs(jnp.cos(xh))
        cw = jnp.abs(jnp.cos(xw))
        out_ref[k] = (off + (ch * float(n - 1)).astype(jnp.int32) * n
                      + (cw * float(n - 1)).astype(jnp.int32))
        k += 1
        off += n * n


def _idx_tc(hc, wc):
    return pl.pallas_call(
        _idx_body,
        grid=(8,),
        in_specs=[pl.BlockSpec((_H // 8, _W), lambda i: (i, 0)),
                  pl.BlockSpec((_H // 8, _W), lambda i: (i, 0))],
        out_specs=pl.BlockSpec((_NUM_LOOKUPS, _H // 8, _W),
                               lambda i: (0, i, 0)),
        out_shape=jax.ShapeDtypeStruct((_NUM_LOOKUPS, _H, _W), jnp.int32),
    )(hc, wc)


def _sc_gather(table, idx):
    mesh = plsc.VectorSubcoreMesh(core_axis_name="c", subcore_axis_name="s")

    @functools.partial(
        pl.kernel,
        mesh=mesh,
        compiler_params=pltpu.CompilerParams(use_tc_tiling_on_sc=False),
        out_type=jax.ShapeDtypeStruct((_H, _W, _NUM_LOOKUPS * _F),
                                      jnp.float32),
        scratch_types=[
            pltpu.VMEM((2, _NUM_LOOKUPS, _SUPR, _W), jnp.int32),
            pltpu.VMEM((2, _NUM_LOOKUPS, _CHUNK, _F), jnp.float32),
            pltpu.SemaphoreType.DMA,
            pltpu.SemaphoreType.DMA,
            pltpu.SemaphoreType.DMA,
        ],
    )
    def k(table_hbm, idx_hbm, out_hbm, idx_v, bufs_v, sem_i, sem_g, sem_w):
        w = lax.axis_index("s") * 2 + lax.axis_index("c")
        r0 = w * _ROWS_W

        def load_idx(urow, slot):
            pltpu.async_copy(
                idx_hbm.at[:, pl.ds(r0 + urow, _SUPR), :], idx_v.at[slot],
                sem_i)

        def idx_slice(islot, u, j):
            return idx_v.at[islot, j, (u // _QPR) % _SUPR,
                            pl.ds((u % _QPR) * _CHUNK, _CHUNK)]

        def fire_gathers(islot, u, bslot):
            for j in range(_NUM_LOOKUPS):
                pltpu.async_copy(
                    table_hbm.at[idx_slice(islot, u, j)],
                    bufs_v.at[bslot, j], sem_g)

        def drain_gathers(islot, u, bslot):
            for j in range(_NUM_LOOKUPS):
                pltpu.make_async_copy(
                    table_hbm.at[idx_slice(islot, u, j)],
                    bufs_v.at[bslot, j], sem_g).wait()

        def out_slice(u, j):
            return out_hbm.at[r0 + u // _QPR,
                              pl.ds((u % _QPR) * _CHUNK, _CHUNK),
                              pl.ds(j * _F, _F)]

        def fire_writes(u, bslot):
            for j in range(_NUM_LOOKUPS):
                pltpu.async_copy(bufs_v.at[bslot, j], out_slice(u, j),
                                 sem_w)

        def drain_writes(u, bslot):
            for j in range(_NUM_LOOKUPS):
                pltpu.make_async_copy(bufs_v.at[bslot, j], out_slice(u, j),
                                      sem_w).wait()

        load_idx(0, 0)
        pltpu.make_async_copy(
            idx_hbm.at[:, pl.ds(r0, _SUPR), :], idx_v.at[0], sem_i).wait()
        load_idx(_SUPR, 1)
        fire_gathers(0, 0, 0)

        @pl.loop(0, _PER_W)
        def _(u):
            nxt = u + 1
            snxt = nxt // _SUP
            at_boundary = jnp.logical_and(nxt % _SUP == 0, nxt < _PER_W)

            @pl.when(at_boundary)
            def _():
                pltpu.make_async_copy(
                    idx_hbm.at[:, pl.ds(r0 + nxt // _QPR, _SUPR), :],
                    idx_v.at[snxt % 2], sem_i).wait()

            drain_gathers(u // _SUP % 2, u, u % 2)

            @pl.when(jnp.logical_and(at_boundary, nxt + _SUP < _PER_W))
            def _():
                load_idx((nxt + _SUP) // _QPR, (snxt + 1) % 2)

            @pl.when(u >= 1)
            def _():
                drain_writes(u - 1, (u - 1) % 2)

            @pl.when(nxt < _PER_W)
            def _():
                fire_gathers(snxt % 2, nxt, nxt % 2)

            fire_writes(u, u % 2)

        drain_writes(_PER_W - 1, (_PER_W - 1) % 2)

    return k(table, idx)


def kernel(c, g0, g1, g2, g3, g4):
    hc = c[:, :, 0]
    wc = c[:, :, 1]
    idx = _idx_tc(hc, wc)
    table = jnp.concatenate(
        [jnp.moveaxis(g, 0, -1).reshape(-1, _F)
         for g in (g0, g1, g2, g3, g4)], axis=0)
    return _sc_gather(table, idx)


# speedup vs baseline: 1.0171x; 1.0171x over previous
import functools
import math

import jax
import jax.numpy as jnp
from jax import lax
from jax.experimental import pallas as pl
from jax.experimental.pallas import tpu as pltpu
from jax.experimental.pallas import tpu_sc as plsc

_F = 16
_H = 512
_W = 512
_N = _H * _W
_GRID_N = (32, 64, 128, 256, 512)
_NUM_LOOKUPS = 9

_NW = 32
_CHUNK = 128
_QPR = _W // _CHUNK
_NCHUNKS = _N // _CHUNK
_PER_W = _NCHUNKS // _NW
_ROWS_W = _H // _NW
_SUPR = 2
_SUP = _SUPR * _QPR


def _idx_body(hc_ref, wc_ref, out_ref):
    hc = hc_ref[...]
    wc = wc_ref[...]
    n0 = _GRID_N[0]
    ih = (hc * float(n0 - 1)).astype(jnp.int32)
    iw = (wc * float(n0 - 1)).astype(jnp.int32)
    out_ref[0] = ih * n0 + iw
    off = n0 * n0
    k = 1
    for i in range(1, 5):
        n = _GRID_N[i]
        m = math.pi * (2.0 ** i)
        xh = hc * m
        xw = wc * m
        sh = jnp.abs(jnp.sin(xh))
        sw = jnp.abs(jnp.sin(xw))
        out_ref[k] = (off + (sh * float(n - 1)).astype(jnp.int32) * n
                      + (sw * float(n - 1)).astype(jnp.int32))
        k += 1
        ch = jnp.abs(jnp.cos(xh))
        cw = jnp.abs(jnp.cos(xw))
        out_ref[k] = (off + (ch * float(n - 1)).astype(jnp.int32) * n
                      + (cw * float(n - 1)).astype(jnp.int32))
        k += 1
        off += n * n


def _idx_tc(hc, wc):
    return pl.pallas_call(
        _idx_body,
        grid=(8,),
        in_specs=[pl.BlockSpec((_H // 8, _W), lambda i: (i, 0)),
                  pl.BlockSpec((_H // 8, _W), lambda i: (i, 0))],
        out_specs=pl.BlockSpec((_NUM_LOOKUPS, _H // 8, _W),
                               lambda i: (0, i, 0)),
        out_shape=jax.ShapeDtypeStruct((_NUM_LOOKUPS, _H, _W), jnp.int32),
    )(hc, wc)


def _sc_gather(table, idx):
    mesh = plsc.VectorSubcoreMesh(core_axis_name="c", subcore_axis_name="s")

    @functools.partial(
        pl.kernel,
        mesh=mesh,
        compiler_params=pltpu.CompilerParams(use_tc_tiling_on_sc=False),
        out_type=jax.ShapeDtypeStruct((_H, _W, _NUM_LOOKUPS * _F),
                                      jnp.float32),
        scratch_types=[
            pltpu.VMEM((2, _NUM_LOOKUPS, _SUPR, _W), jnp.int32),
            pltpu.VMEM((2, _NUM_LOOKUPS, _CHUNK, _F), jnp.float32),
            pltpu.SemaphoreType.DMA,
            pltpu.SemaphoreType.DMA,
            pltpu.SemaphoreType.DMA,
        ],
    )
    def k(table_hbm, idx_hbm, out_hbm, idx_v, bufs_v, sem_i, sem_g, sem_w):
        w = lax.axis_index("s") * 2 + lax.axis_index("c")
        r0 = w * _ROWS_W

        def load_idx(urow, slot):
            pltpu.async_copy(
                idx_hbm.at[:, pl.ds(r0 + urow, _SUPR), :], idx_v.at[slot],
                sem_i)

        def idx_slice(islot, u, j):
            return idx_v.at[islot, j, (u // _QPR) % _SUPR,
                            pl.ds((u % _QPR) * _CHUNK, _CHUNK)]

        def fire_gathers(islot, u, bslot):
            for j in range(_NUM_LOOKUPS):
                pltpu.async_copy(
                    table_hbm.at[idx_slice(islot, u, j)],
                    bufs_v.at[bslot, j], sem_g)

        def drain_gathers(islot, u, bslot):
            for j in range(_NUM_LOOKUPS):
                pltpu.make_async_copy(
                    table_hbm.at[idx_slice(islot, u, j)],
                    bufs_v.at[bslot, j], sem_g).wait()

        def out_slice(u, j):
            return out_hbm.at[r0 + u // _QPR,
                              pl.ds((u % _QPR) * _CHUNK, _CHUNK),
                              pl.ds(j * _F, _F)]

        def fire_writes(u, bslot):
            for j in range(_NUM_LOOKUPS):
                pltpu.async_copy(bufs_v.at[bslot, j], out_slice(u, j),
                                 sem_w)

        def drain_writes(u, bslot):
            for j in range(_NUM_LOOKUPS):
                pltpu.make_async_copy(bufs_v.at[bslot, j], out_slice(u, j),
                                      sem_w).wait()

        load_idx(0, 0)
        pltpu.make_async_copy(
            idx_hbm.at[:, pl.ds(r0, _SUPR), :], idx_v.at[0], sem_i).wait()
        load_idx(_SUPR, 1)
        fire_gathers(0, 0, 0)

        @pl.loop(0, _PER_W)
        def _(u):
            nxt = u + 1
            snxt = nxt // _SUP
            at_boundary = jnp.logical_and(nxt % _SUP == 0, nxt < _PER_W)

            @pl.when(at_boundary)
            def _():
                pltpu.make_async_copy(
                    idx_hbm.at[:, pl.ds(r0 + nxt // _QPR, _SUPR), :],
                    idx_v.at[snxt % 2], sem_i).wait()

            drain_gathers(u // _SUP % 2, u, u % 2)

            @pl.when(jnp.logical_and(at_boundary, nxt + _SUP < _PER_W))
            def _():
                load_idx((nxt + _SUP) // _QPR, (snxt + 1) % 2)

            @pl.when(u >= 1)
            def _():
                drain_writes(u - 1, (u - 1) % 2)

            @pl.when(nxt < _PER_W)
            def _():
                fire_gathers(snxt % 2, nxt, nxt % 2)

            fire_writes(u, u % 2)

        drain_writes(_PER_W - 1, (_PER_W - 1) % 2)

    return k(table, idx)


def kernel(c, g0, g1, g2, g3, g4):
    hc = c[:, :, 0]
    wc = c[:, :, 1]
    idx = _idx_tc(hc, wc)
    table = jnp.concatenate(
        [jnp.moveaxis(g, 0, -1).reshape(-1, 128)
         for g in (g0, g1, g2, g3, g4)], axis=0).reshape(-1, _F)
    return _sc_gather(table, idx)
